# Initial kernel scaffold; baseline (speedup 1.0000x reference)
import jax
import jax.numpy as jnp
from jax.experimental import pallas as pl


def kernel(x, edge_index, WX, WZ, Walpha):
    raise NotImplementedError("write your pallas kernel here")


# speedup vs baseline: 3.1841x; 3.1841x over previous
import jax
import jax.numpy as jnp
from jax import lax
from jax.experimental import pallas as pl
from jax.experimental.pallas import tpu as pltpu
from jax.experimental.pallas import tpu_sc as plsc

N = 10000
E = 320000
D = 128
NC = 2
NS = 16
NW = NC * NS
EPW = E // NW
CH = 128
NSTAGE = 2
SPC = 40
NCHUNK = NSTAGE * SPC
EPW_PAD = NCHUNK * CH
RPT = 640
N_PAD = NS * RPT
CNT_W = 128
BLK = 400


def _deg_body(e1p, ones_in, zeros_in, cnt_out, idx_v, stage_v, hist_sh):
    c = lax.axis_index("c")
    s = lax.axis_index("s")
    w = s * NC + c
    pltpu.sync_copy(zeros_in, stage_v)
    for k in range(RPT // CH):
        pltpu.sync_copy(stage_v, hist_sh.at[pl.ds(s * RPT + k * CH, CH)])
    pltpu.sync_copy(e1p.at[w], idx_v)
    pltpu.sync_copy(ones_in, stage_v)
    plsc.subcore_barrier()
    for j in range(NCHUNK):
        pltpu.sync_copy(stage_v, hist_sh.at[idx_v.at[j]], add=True)
    plsc.subcore_barrier()
    for k in range(RPT // CH):
        slab = pl.ds(s * RPT + k * CH, CH)
        pltpu.sync_copy(hist_sh.at[slab], stage_v)
        pltpu.sync_copy(stage_v, cnt_out.at[c, slab])


def _msg_body(y, e0p, e1p, zeros_in, p_out,
              idx0_v, idx1_v, rows_a, rows_b, accum_sh, sem_a, sem_b):
    c = lax.axis_index("c")
    s = lax.axis_index("s")
    w = s * NC + c
    pltpu.sync_copy(zeros_in, rows_a)
    for k in range(RPT // CH):
        pltpu.sync_copy(rows_a, accum_sh.at[pl.ds(s * RPT + k * CH, CH)])
    plsc.subcore_barrier()
    rows = (rows_a, rows_b)
    sems = (sem_a, sem_b)
    descs = [None] * NCHUNK
    for st in range(NSTAGE):
        pltpu.sync_copy(e0p.at[w, pl.ds(st * SPC, SPC)], idx0_v)
        pltpu.sync_copy(e1p.at[w, pl.ds(st * SPC, SPC)], idx1_v)
        for jj in range(SPC):
            j = st * SPC + jj
            if jj == 0:
                descs[j] = pltpu.async_copy(
                    y.at[idx1_v.at[0]], rows[j % 2], sems[j % 2])
            if jj + 1 < SPC:
                descs[j + 1] = pltpu.async_copy(
                    y.at[idx1_v.at[jj + 1]], rows[(j + 1) % 2],
                    sems[(j + 1) % 2])
            descs[j].wait()
            pltpu.sync_copy(rows[j % 2], accum_sh.at[idx0_v.at[jj]], add=True)
    plsc.subcore_barrier()
    for k in range(RPT // CH):
        slab = pl.ds(s * RPT + k * CH, CH)
        pltpu.sync_copy(accum_sh.at[slab], rows_a)
        pltpu.sync_copy(rows_a, p_out.at[c, slab])


def _xw_body(cnt_ref, x_ref, wx_ref, y_ref):
    cnt = cnt_ref[...]
    deg = cnt[0, :, 0] + cnt[1, :, 0] + 1.0
    d = lax.rsqrt(deg)
    xp = lax.dot_general(x_ref[...], wx_ref[...], (((1,), (1,)), ((), ())),
                         preferred_element_type=jnp.float32)
    y_ref[...] = d[:, None] * xp


def _out_body(cnt_ref, y_ref, p_ref, o_ref):
    cnt = cnt_ref[...]
    deg = cnt[0, :, 0] + cnt[1, :, 0] + 1.0
    d = lax.rsqrt(deg)
    o_ref[...] = d[:, None] * (y_ref[...] + p_ref[0] + p_ref[1])


_mesh = plsc.VectorSubcoreMesh(core_axis_name="c", subcore_axis_name="s")

_deg_call = pl.kernel(
    _deg_body,
    out_type=jax.ShapeDtypeStruct((NC, N_PAD, CNT_W), jnp.float32),
    mesh=_mesh,
    scratch_types=[
        pltpu.VMEM((NCHUNK, CH), jnp.int32),
        pltpu.VMEM((CH, CNT_W), jnp.float32),
        pltpu.VMEM_SHARED((N_PAD, CNT_W), jnp.float32),
    ],
)

_msg_call = pl.kernel(
    _msg_body,
    out_type=jax.ShapeDtypeStruct((NC, N_PAD, D), jnp.float32),
    mesh=_mesh,
    scratch_types=[
        pltpu.VMEM((SPC, CH), jnp.int32),
        pltpu.VMEM((SPC, CH), jnp.int32),
        pltpu.VMEM((CH, D), jnp.float32),
        pltpu.VMEM((CH, D), jnp.float32),
        pltpu.VMEM_SHARED((N_PAD, D), jnp.float32),
        pltpu.SemaphoreType.DMA,
        pltpu.SemaphoreType.DMA,
    ],
)


def kernel(x, edge_index, WX, WZ, Walpha):
    e0 = edge_index[0].reshape(NW, EPW)
    e1 = edge_index[1].reshape(NW, EPW)
    pad = ((0, 0), (0, EPW_PAD - EPW))
    e0p = jnp.pad(e0, pad, constant_values=N).reshape(NW, NCHUNK, CH)
    e1p_s = jnp.pad(e1, pad, constant_values=N).reshape(NW, NCHUNK, CH)
    e1p_g = jnp.pad(e1, pad, constant_values=0).reshape(NW, NCHUNK, CH)

    onehot = jnp.zeros((CH, CNT_W), jnp.float32).at[:, 0].set(1.0)
    zeros16 = jnp.zeros((CH, CNT_W), jnp.float32)
    zeros128 = jnp.zeros((CH, D), jnp.float32)

    counts = _deg_call(e1p_s, onehot, zeros16)

    y = pl.pallas_call(
        _xw_body,
        grid=(N // BLK,),
        in_specs=[pl.BlockSpec((NC, BLK, CNT_W), lambda i: (0, i, 0)),
                  pl.BlockSpec((BLK, D), lambda i: (i, 0)),
                  pl.BlockSpec((D, D), lambda i: (0, 0))],
        out_specs=pl.BlockSpec((BLK, D), lambda i: (i, 0)),
        out_shape=jax.ShapeDtypeStruct((N, D), jnp.float32),
    )(counts, x, WX)

    p = _msg_call(y, e0p, e1p_g, zeros128)

    out = pl.pallas_call(
        _out_body,
        grid=(N // BLK,),
        in_specs=[pl.BlockSpec((NC, BLK, CNT_W), lambda i: (0, i, 0)),
                  pl.BlockSpec((BLK, D), lambda i: (i, 0)),
                  pl.BlockSpec((NC, BLK, D), lambda i: (0, i, 0))],
        out_specs=pl.BlockSpec((BLK, D), lambda i: (i, 0)),
        out_shape=jax.ShapeDtypeStruct((N, D), jnp.float32),
    )(counts, y, p)
    return out


# speedup vs baseline: 3.1846x; 1.0002x over previous
import jax
import jax.numpy as jnp
from jax import lax
from jax.experimental import pallas as pl
from jax.experimental.pallas import tpu as pltpu
from jax.experimental.pallas import tpu_sc as plsc

N = 10000
E = 320000
D = 128
NC = 2
NS = 16
NW = NC * NS
EPW = E // NW
RPT = 640
N_PAD = NS * RPT
CNT_W = 128
BLK = 400

CH1 = 128
NCHUNK1 = 80
EPW_PAD1 = NCHUNK1 * CH1

CH3 = 64
NCHUNK3 = 160
EPW_PAD3 = NCHUNK3 * CH3
NBUF = 3
NSTAGE = 2
SPC = NCHUNK3 // NSTAGE


def _deg_body(e1p, ones_in, zeros_in, cnt_out, idx_v, stage_v, hist_sh):
    c = lax.axis_index("c")
    s = lax.axis_index("s")
    w = s * NC + c
    pltpu.sync_copy(zeros_in, stage_v)
    for k in range(RPT // CH1):
        pltpu.sync_copy(stage_v, hist_sh.at[pl.ds(s * RPT + k * CH1, CH1)])
    pltpu.sync_copy(e1p.at[w], idx_v)
    pltpu.sync_copy(ones_in, stage_v)
    plsc.subcore_barrier()
    for j in range(NCHUNK1):
        pltpu.sync_copy(stage_v, hist_sh.at[idx_v.at[j]], add=True)
    plsc.subcore_barrier()
    for k in range(RPT // CH1):
        slab = pl.ds(s * RPT + k * CH1, CH1)
        pltpu.sync_copy(hist_sh.at[slab], stage_v)
        pltpu.sync_copy(stage_v, cnt_out.at[c, slab])


def _msg_body(y, e0p, e1p, zeros_in, p_out,
              idx0_v, idx1_v, r0, r1, r2, accum_sh, s0, s1, s2):
    c = lax.axis_index("c")
    s = lax.axis_index("s")
    w = s * NC + c
    rows = (r0, r1, r2)
    sems = (s0, s1, s2)
    pltpu.sync_copy(zeros_in, r0)
    pltpu.sync_copy(zeros_in, r1)
    for k in range(RPT // (2 * CH3)):
        pltpu.sync_copy(r0, accum_sh.at[pl.ds(s * RPT + 2 * k * CH3, CH3)])
        pltpu.sync_copy(r1, accum_sh.at[pl.ds(s * RPT + (2 * k + 1) * CH3,
                                              CH3)])
    plsc.subcore_barrier()
    descs = [None] * NCHUNK3
    for st in range(NSTAGE):
        pltpu.sync_copy(e0p.at[w, pl.ds(st * SPC, SPC)], idx0_v)
        pltpu.sync_copy(e1p.at[w, pl.ds(st * SPC, SPC)], idx1_v)
        for jj in range(SPC):
            j = st * SPC + jj
            if jj == 0:
                for b in range(NBUF - 1):
                    descs[j + b] = pltpu.async_copy(
                        y.at[idx1_v.at[b]], rows[(j + b) % NBUF],
                        sems[(j + b) % NBUF])
            jj_n = jj + NBUF - 1
            if jj_n < SPC:
                jn = j + NBUF - 1
                descs[jn] = pltpu.async_copy(
                    y.at[idx1_v.at[jj_n]], rows[jn % NBUF], sems[jn % NBUF])
            descs[j].wait()
            pltpu.sync_copy(rows[j % NBUF], accum_sh.at[idx0_v.at[jj]],
                            add=True)
    plsc.subcore_barrier()
    for k in range(RPT // (2 * CH3)):
        slab0 = pl.ds(s * RPT + 2 * k * CH3, CH3)
        slab1 = pl.ds(s * RPT + (2 * k + 1) * CH3, CH3)
        pltpu.sync_copy(accum_sh.at[slab0], r0)
        pltpu.sync_copy(accum_sh.at[slab1], r1)
        pltpu.sync_copy(r0, p_out.at[c, slab0])
        pltpu.sync_copy(r1, p_out.at[c, slab1])


def _mm_body(x_ref, wx_ref, xp_ref):
    xp_ref[...] = lax.dot_general(
        x_ref[...], wx_ref[...], (((1,), (1,)), ((), ())),
        preferred_element_type=jnp.float32)


def _scale_body(cnt_ref, xp_ref, y_ref):
    cnt = cnt_ref[...]
    deg = cnt[0, :, 0] + cnt[1, :, 0] + 1.0
    d = lax.rsqrt(deg)
    y_ref[...] = d[:, None] * xp_ref[...]


def _out_body(cnt_ref, y_ref, p_ref, o_ref):
    cnt = cnt_ref[...]
    deg = cnt[0, :, 0] + cnt[1, :, 0] + 1.0
    d = lax.rsqrt(deg)
    o_ref[...] = d[:, None] * (y_ref[...] + p_ref[0] + p_ref[1])


_mesh = plsc.VectorSubcoreMesh(core_axis_name="c", subcore_axis_name="s")

_deg_call = pl.kernel(
    _deg_body,
    out_type=jax.ShapeDtypeStruct((NC, N_PAD, CNT_W), jnp.float32),
    mesh=_mesh,
    scratch_types=[
        pltpu.VMEM((NCHUNK1, CH1), jnp.int32),
        pltpu.VMEM((CH1, CNT_W), jnp.float32),
        pltpu.VMEM_SHARED((N_PAD, CNT_W), jnp.float32),
    ],
)

_msg_call = pl.kernel(
    _msg_body,
    out_type=jax.ShapeDtypeStruct((NC, N_PAD, D), jnp.float32),
    mesh=_mesh,
    scratch_types=[
        pltpu.VMEM((SPC, CH3), jnp.int32),
        pltpu.VMEM((SPC, CH3), jnp.int32),
        pltpu.VMEM((CH3, D), jnp.float32),
        pltpu.VMEM((CH3, D), jnp.float32),
        pltpu.VMEM((CH3, D), jnp.float32),
        pltpu.VMEM_SHARED((N_PAD, D), jnp.float32),
        pltpu.SemaphoreType.DMA,
        pltpu.SemaphoreType.DMA,
        pltpu.SemaphoreType.DMA,
    ],
)


def kernel(x, edge_index, WX, WZ, Walpha):
    e0 = edge_index[0].reshape(NW, EPW)
    e1 = edge_index[1].reshape(NW, EPW)
    pad1 = ((0, 0), (0, EPW_PAD1 - EPW))
    pad3 = ((0, 0), (0, EPW_PAD3 - EPW))
    e1p_s = jnp.pad(e1, pad1, constant_values=N).reshape(NW, NCHUNK1, CH1)
    e0p = jnp.pad(e0, pad3, constant_values=N).reshape(NW, NCHUNK3, CH3)
    e1p_g = jnp.pad(e1, pad3, constant_values=0).reshape(NW, NCHUNK3, CH3)

    onehot = jnp.zeros((CH1, CNT_W), jnp.float32).at[:, 0].set(1.0)
    zeros_cnt = jnp.zeros((CH1, CNT_W), jnp.float32)
    zeros_row = jnp.zeros((CH3, D), jnp.float32)

    counts = _deg_call(e1p_s, onehot, zeros_cnt)

    xp = pl.pallas_call(
        _mm_body,
        grid=(N // BLK,),
        in_specs=[pl.BlockSpec((BLK, D), lambda i: (i, 0)),
                  pl.BlockSpec((D, D), lambda i: (0, 0))],
        out_specs=pl.BlockSpec((BLK, D), lambda i: (i, 0)),
        out_shape=jax.ShapeDtypeStruct((N, D), jnp.float32),
    )(x, WX)

    y = pl.pallas_call(
        _scale_body,
        grid=(N // BLK,),
        in_specs=[pl.BlockSpec((NC, BLK, CNT_W), lambda i: (0, i, 0)),
                  pl.BlockSpec((BLK, D), lambda i: (i, 0))],
        out_specs=pl.BlockSpec((BLK, D), lambda i: (i, 0)),
        out_shape=jax.ShapeDtypeStruct((N, D), jnp.float32),
    )(counts, xp)

    p = _msg_call(y, e0p, e1p_g, zeros_row)

    out = pl.pallas_call(
        _out_body,
        grid=(N // BLK,),
        in_specs=[pl.BlockSpec((NC, BLK, CNT_W), lambda i: (0, i, 0)),
                  pl.BlockSpec((BLK, D), lambda i: (i, 0)),
                  pl.BlockSpec((NC, BLK, D), lambda i: (0, i, 0))],
        out_specs=pl.BlockSpec((BLK, D), lambda i: (i, 0)),
        out_shape=jax.ShapeDtypeStruct((N, D), jnp.float32),
    )(counts, y, p)
    return out
